# Initial kernel scaffold; baseline (speedup 1.0000x reference)
import jax
import jax.numpy as jnp
from jax.experimental import pallas as pl


def kernel(x, edge_index, W1, b1, W2, b2, Wa1, ba1, Wa2, ba2, Wv1, bv1, Wv2, bv2):
    raise NotImplementedError("write your pallas kernel here")


# speedup vs baseline: 4.0357x; 4.0357x over previous
import jax
import jax.numpy as jnp
from jax import lax
from jax.experimental import pallas as pl
from jax.experimental.pallas import tpu as pltpu
from jax.experimental.pallas import tpu_sc as plsc

N_NODES = 10000
N_EDGES = 320000
D = 128
DH = D // 2
D_STREAM = 256
N_ACTIONS = 64

NC, NS = 2, 16
CHUNK = 128
RPW = 160
NROWS_TOT = RPW * NS
E_PAD = NROWS_TOT * CHUNK
DUMMY = N_NODES
AGG_ROWS = 10112
ZROWS = AGG_ROWS // NS
DEG_RPW = RPW // NC

_MESH = plsc.VectorSubcoreMesh(core_axis_name="c", subcore_axis_name="s")


def _sc1_body(x_hbm, srcq, dstq, z64, z16, ones_hbm, agg_out, deg_out,
              src_v, dst_v, rows_v, ones_v, agg_sh, deg_sh, sem):
    c = lax.axis_index("c")
    s = lax.axis_index("s")
    pltpu.sync_copy(z64.at[pl.ds(s * ZROWS, ZROWS)],
                    agg_sh.at[pl.ds(s * ZROWS, ZROWS)])
    pltpu.sync_copy(z16.at[pl.ds(s * ZROWS, ZROWS)],
                    deg_sh.at[pl.ds(s * ZROWS, ZROWS)])
    pltpu.sync_copy(ones_hbm, ones_v)
    pltpu.sync_copy(srcq.at[c, pl.ds(s * RPW, RPW)], src_v)
    pltpu.sync_copy(dstq.at[pl.ds(s * RPW, RPW)], dst_v)
    plsc.subcore_barrier()

    deg_lo = c * DEG_RPW

    def step(j, carry):
        pltpu.async_copy(x_hbm.at[src_v.at[j]], rows_v, sem).wait()
        pltpu.sync_copy(rows_v, agg_sh.at[dst_v.at[j]], add=True)

        @pl.when((j >= deg_lo) & (j < deg_lo + DEG_RPW))
        def _():
            pltpu.sync_copy(ones_v, deg_sh.at[dst_v.at[j]], add=True)

        return carry

    lax.fori_loop(0, RPW, step, 0)
    plsc.subcore_barrier()
    pltpu.sync_copy(agg_sh.at[pl.ds(s * ZROWS, ZROWS)],
                    agg_out.at[c, pl.ds(s * ZROWS, ZROWS)])
    pltpu.sync_copy(deg_sh.at[pl.ds(s * ZROWS, ZROWS)],
                    deg_out.at[c, pl.ds(s * ZROWS, ZROWS)])


def _sc2_body(h_hbm, srcq, dstq, z64, agg_out,
              src_v, dst_v, rows_v, agg_sh, sem):
    c = lax.axis_index("c")
    s = lax.axis_index("s")
    pltpu.sync_copy(z64.at[pl.ds(s * ZROWS, ZROWS)],
                    agg_sh.at[pl.ds(s * ZROWS, ZROWS)])
    pltpu.sync_copy(srcq.at[c, pl.ds(s * RPW, RPW)], src_v)
    pltpu.sync_copy(dstq.at[pl.ds(s * RPW, RPW)], dst_v)
    plsc.subcore_barrier()

    def step(j, carry):
        pltpu.async_copy(h_hbm.at[src_v.at[j]], rows_v, sem).wait()
        pltpu.sync_copy(rows_v, agg_sh.at[dst_v.at[j]], add=True)
        return carry

    lax.fori_loop(0, RPW, step, 0)
    plsc.subcore_barrier()
    pltpu.sync_copy(agg_sh.at[pl.ds(s * ZROWS, ZROWS)],
                    agg_out.at[c, pl.ds(s * ZROWS, ZROWS)])


_sc_layer1 = pl.kernel(
    _sc1_body,
    out_type=[
        jax.ShapeDtypeStruct((NC, AGG_ROWS, DH), jnp.float32),
        jax.ShapeDtypeStruct((NC, AGG_ROWS, 16), jnp.float32),
    ],
    mesh=_MESH,
    compiler_params=pltpu.CompilerParams(use_tc_tiling_on_sc=False),
    scratch_types=[
        pltpu.VMEM((RPW, CHUNK), jnp.int32),
        pltpu.VMEM((RPW, CHUNK), jnp.int32),
        pltpu.VMEM((CHUNK, DH), jnp.float32),
        pltpu.VMEM((CHUNK, 16), jnp.float32),
        pltpu.VMEM_SHARED((AGG_ROWS, DH), jnp.float32),
        pltpu.VMEM_SHARED((AGG_ROWS, 16), jnp.float32),
        pltpu.SemaphoreType.DMA,
    ],
)

_sc_layer2 = pl.kernel(
    _sc2_body,
    out_type=[jax.ShapeDtypeStruct((NC, AGG_ROWS, DH), jnp.float32)],
    mesh=_MESH,
    compiler_params=pltpu.CompilerParams(use_tc_tiling_on_sc=False),
    scratch_types=[
        pltpu.VMEM((RPW, CHUNK), jnp.int32),
        pltpu.VMEM((RPW, CHUNK), jnp.int32),
        pltpu.VMEM((CHUNK, DH), jnp.float32),
        pltpu.VMEM_SHARED((AGG_ROWS, DH), jnp.float32),
        pltpu.SemaphoreType.DMA,
    ],
)

BLK = 2000
GRID = N_NODES // BLK


def _tc1_body(agg_ref, deg_ref, w_ref, b_ref, o_ref):
    a = jnp.concatenate([agg_ref[0], agg_ref[1]], axis=1)
    deg = deg_ref[0] + deg_ref[1]
    deg = jnp.maximum(deg[:, 0:1], 1.0)
    h = jnp.maximum(
        jnp.dot(a / deg, w_ref[...], preferred_element_type=jnp.float32)
        + b_ref[...], 0.0)
    o_ref[0, :, :] = h[:, :DH]
    o_ref[1, :, :] = h[:, DH:]


def _tc2_body(agg_ref, deg_ref, w2, b2, wa1, ba1, wa2, ba2,
              wv1, bv1, wv2, bv2, q_ref, acc_ref):
    i = pl.program_id(0)
    a = jnp.concatenate([agg_ref[0], agg_ref[1]], axis=1)
    deg = deg_ref[0] + deg_ref[1]
    deg = jnp.maximum(deg[:, 0:1], 1.0)
    h = jnp.maximum(
        jnp.dot(a / deg, w2[...], preferred_element_type=jnp.float32)
        + b2[...], 0.0)
    part = jnp.sum(h, axis=0, keepdims=True)

    @pl.when(i == 0)
    def _():
        acc_ref[...] = part

    @pl.when(i > 0)
    def _():
        acc_ref[...] = acc_ref[...] + part

    @pl.when(i == pl.num_programs(0) - 1)
    def _():
        ge = acc_ref[...] * (1.0 / N_NODES)
        adv = jnp.maximum(
            jnp.dot(ge, wa1[...], preferred_element_type=jnp.float32)
            + ba1[...], 0.0)
        aq = (jnp.dot(adv, wa2[...], preferred_element_type=jnp.float32)
              + ba2[...])
        val = jnp.maximum(
            jnp.dot(ge, wv1[...], preferred_element_type=jnp.float32)
            + bv1[...], 0.0)
        v = (jnp.dot(val, wv2[...], preferred_element_type=jnp.float32)
             + bv2[...])
        q_ref[...] = v + aq - jnp.mean(aq)


_tc_layer1 = pl.pallas_call(
    _tc1_body,
    grid=(GRID,),
    in_specs=[
        pl.BlockSpec((2, BLK, DH), lambda i: (0, i, 0)),
        pl.BlockSpec((2, BLK, 16), lambda i: (0, i, 0)),
        pl.BlockSpec((D, D), lambda i: (0, 0)),
        pl.BlockSpec((1, D), lambda i: (0, 0)),
    ],
    out_specs=pl.BlockSpec((2, BLK, DH), lambda i: (0, i, 0)),
    out_shape=jax.ShapeDtypeStruct((2, N_NODES, DH), jnp.float32),
)

_tc_head = pl.pallas_call(
    _tc2_body,
    grid=(GRID,),
    in_specs=[
        pl.BlockSpec((2, BLK, DH), lambda i: (0, i, 0)),
        pl.BlockSpec((2, BLK, 16), lambda i: (0, i, 0)),
        pl.BlockSpec((D, D), lambda i: (0, 0)),
        pl.BlockSpec((1, D), lambda i: (0, 0)),
        pl.BlockSpec((D, D_STREAM), lambda i: (0, 0)),
        pl.BlockSpec((1, D_STREAM), lambda i: (0, 0)),
        pl.BlockSpec((D_STREAM, N_ACTIONS), lambda i: (0, 0)),
        pl.BlockSpec((1, N_ACTIONS), lambda i: (0, 0)),
        pl.BlockSpec((D, D_STREAM), lambda i: (0, 0)),
        pl.BlockSpec((1, D_STREAM), lambda i: (0, 0)),
        pl.BlockSpec((D_STREAM, 1), lambda i: (0, 0)),
        pl.BlockSpec((1, 1), lambda i: (0, 0)),
    ],
    out_specs=pl.BlockSpec((1, N_ACTIONS), lambda i: (0, 0)),
    out_shape=jax.ShapeDtypeStruct((1, N_ACTIONS), jnp.float32),
    scratch_shapes=[pltpu.VMEM((1, D), jnp.float32)],
)


def kernel(x, edge_index, W1, b1, W2, b2, Wa1, ba1, Wa2, ba2,
           Wv1, bv1, Wv2, bv2):
    src = edge_index[0].astype(jnp.int32)
    dst = edge_index[1].astype(jnp.int32)
    pad = E_PAD - N_EDGES
    srcq = jnp.concatenate([src, jnp.zeros((pad,), jnp.int32)])
    dstq = jnp.concatenate([dst, jnp.full((pad,), DUMMY, jnp.int32)])
    srcq = srcq.reshape(NROWS_TOT, CHUNK)
    dstq = dstq.reshape(NROWS_TOT, CHUNK)
    srcq2 = jnp.stack([srcq, srcq + N_NODES])
    z64 = jnp.zeros((AGG_ROWS, DH), jnp.float32)
    z16 = jnp.zeros((AGG_ROWS, 16), jnp.float32)
    ones16 = jnp.ones((CHUNK, 16), jnp.float32)
    x_cat = jnp.concatenate([x[:, :DH], x[:, DH:]], axis=0)

    agg1, degm = _sc_layer1(x_cat, srcq2, dstq, z64, z16, ones16)
    h1 = _tc_layer1(agg1, degm, W1, b1.reshape(1, D))
    h1_cat = h1.reshape(2 * N_NODES, DH)
    (agg2,) = _sc_layer2(h1_cat, srcq2, dstq, z64)
    q = _tc_head(agg2, degm, W2, b2.reshape(1, D),
                 Wa1, ba1.reshape(1, D_STREAM), Wa2, ba2.reshape(1, N_ACTIONS),
                 Wv1, bv1.reshape(1, D_STREAM), Wv2, bv2.reshape(1, 1))
    return q


# speedup vs baseline: 5.3465x; 1.3248x over previous
import jax
import jax.numpy as jnp
from jax import lax
from jax.experimental import pallas as pl
from jax.experimental.pallas import tpu as pltpu
from jax.experimental.pallas import tpu_sc as plsc

N_NODES = 10000
N_EDGES = 320000
D = 128
DH = D // 2
D_STREAM = 256
N_ACTIONS = 64

NC, NS = 2, 16
CHUNK = 128
RPW = 160
NROWS_TOT = RPW * NS
E_PAD = NROWS_TOT * CHUNK
DUMMY = N_NODES
AGG_ROWS = 10112
ZROWS = AGG_ROWS // NS
DEG_RPW = RPW // NC

_MESH = plsc.VectorSubcoreMesh(core_axis_name="c", subcore_axis_name="s")


NBUF = 4


def _sc1_body(x_hbm, srcq, dstq, z64, z16, ones_hbm, agg_out, deg_out,
              src_v, dst_v, r0, r1, r2, r3, ones_v, agg_sh, deg_sh,
              g0, g1, g2, g3):
    rows = (r0, r1, r2, r3)
    sems = (g0, g1, g2, g3)
    c = lax.axis_index("c")
    s = lax.axis_index("s")
    pltpu.sync_copy(z64.at[pl.ds(s * ZROWS, ZROWS)],
                    agg_sh.at[pl.ds(s * ZROWS, ZROWS)])
    pltpu.sync_copy(z16.at[pl.ds(s * ZROWS, ZROWS)],
                    deg_sh.at[pl.ds(s * ZROWS, ZROWS)])
    pltpu.sync_copy(ones_hbm, ones_v)
    pltpu.sync_copy(srcq.at[c, pl.ds(s * RPW, RPW)], src_v)
    pltpu.sync_copy(dstq.at[pl.ds(s * RPW, RPW)], dst_v)
    plsc.subcore_barrier()

    deg_lo = c * DEG_RPW
    for b in range(NBUF):
        pltpu.async_copy(x_hbm.at[src_v.at[b]], rows[b], sems[b])

    def group(g, carry):
        for b in range(NBUF):
            j = g * NBUF + b
            pltpu.make_async_copy(x_hbm.at[src_v.at[j]],
                                  rows[b], sems[b]).wait()
            pltpu.sync_copy(rows[b], agg_sh.at[dst_v.at[j]], add=True)

            @pl.when((j >= deg_lo) & (j < deg_lo + DEG_RPW))
            def _():
                pltpu.sync_copy(ones_v, deg_sh.at[dst_v.at[j]], add=True)

            @pl.when(j + NBUF < RPW)
            def _():
                pltpu.async_copy(x_hbm.at[src_v.at[j + NBUF]],
                                 rows[b], sems[b])

        return carry

    lax.fori_loop(0, RPW // NBUF, group, 0)
    plsc.subcore_barrier()
    pltpu.sync_copy(agg_sh.at[pl.ds(s * ZROWS, ZROWS)],
                    agg_out.at[c, pl.ds(s * ZROWS, ZROWS)])
    pltpu.sync_copy(deg_sh.at[pl.ds(s * ZROWS, ZROWS)],
                    deg_out.at[c, pl.ds(s * ZROWS, ZROWS)])


def _sc2_body(h_hbm, srcq, dstq, z64, agg_out,
              src_v, dst_v, r0, r1, r2, r3, agg_sh,
              g0, g1, g2, g3):
    rows = (r0, r1, r2, r3)
    sems = (g0, g1, g2, g3)
    c = lax.axis_index("c")
    s = lax.axis_index("s")
    pltpu.sync_copy(z64.at[pl.ds(s * ZROWS, ZROWS)],
                    agg_sh.at[pl.ds(s * ZROWS, ZROWS)])
    pltpu.sync_copy(srcq.at[c, pl.ds(s * RPW, RPW)], src_v)
    pltpu.sync_copy(dstq.at[pl.ds(s * RPW, RPW)], dst_v)
    plsc.subcore_barrier()

    for b in range(NBUF):
        pltpu.async_copy(h_hbm.at[src_v.at[b]], rows[b], sems[b])

    def group(g, carry):
        for b in range(NBUF):
            j = g * NBUF + b
            pltpu.make_async_copy(h_hbm.at[src_v.at[j]],
                                  rows[b], sems[b]).wait()
            pltpu.sync_copy(rows[b], agg_sh.at[dst_v.at[j]], add=True)

            @pl.when(j + NBUF < RPW)
            def _():
                pltpu.async_copy(h_hbm.at[src_v.at[j + NBUF]],
                                 rows[b], sems[b])

        return carry

    lax.fori_loop(0, RPW // NBUF, group, 0)
    plsc.subcore_barrier()
    pltpu.sync_copy(agg_sh.at[pl.ds(s * ZROWS, ZROWS)],
                    agg_out.at[c, pl.ds(s * ZROWS, ZROWS)])


_sc_layer1 = pl.kernel(
    _sc1_body,
    out_type=[
        jax.ShapeDtypeStruct((NC, AGG_ROWS, DH), jnp.float32),
        jax.ShapeDtypeStruct((NC, AGG_ROWS, 16), jnp.float32),
    ],
    mesh=_MESH,
    compiler_params=pltpu.CompilerParams(use_tc_tiling_on_sc=False),
    scratch_types=[
        pltpu.VMEM((RPW, CHUNK), jnp.int32),
        pltpu.VMEM((RPW, CHUNK), jnp.int32),
        pltpu.VMEM((CHUNK, DH), jnp.float32),
        pltpu.VMEM((CHUNK, DH), jnp.float32),
        pltpu.VMEM((CHUNK, DH), jnp.float32),
        pltpu.VMEM((CHUNK, DH), jnp.float32),
        pltpu.VMEM((CHUNK, 16), jnp.float32),
        pltpu.VMEM_SHARED((AGG_ROWS, DH), jnp.float32),
        pltpu.VMEM_SHARED((AGG_ROWS, 16), jnp.float32),
        pltpu.SemaphoreType.DMA,
        pltpu.SemaphoreType.DMA,
        pltpu.SemaphoreType.DMA,
        pltpu.SemaphoreType.DMA,
    ],
)

_sc_layer2 = pl.kernel(
    _sc2_body,
    out_type=[jax.ShapeDtypeStruct((NC, AGG_ROWS, DH), jnp.float32)],
    mesh=_MESH,
    compiler_params=pltpu.CompilerParams(use_tc_tiling_on_sc=False),
    scratch_types=[
        pltpu.VMEM((RPW, CHUNK), jnp.int32),
        pltpu.VMEM((RPW, CHUNK), jnp.int32),
        pltpu.VMEM((CHUNK, DH), jnp.float32),
        pltpu.VMEM((CHUNK, DH), jnp.float32),
        pltpu.VMEM((CHUNK, DH), jnp.float32),
        pltpu.VMEM((CHUNK, DH), jnp.float32),
        pltpu.VMEM_SHARED((AGG_ROWS, DH), jnp.float32),
        pltpu.SemaphoreType.DMA,
        pltpu.SemaphoreType.DMA,
        pltpu.SemaphoreType.DMA,
        pltpu.SemaphoreType.DMA,
    ],
)

BLK = 2000
GRID = N_NODES // BLK


def _tc1_body(agg_ref, deg_ref, w_ref, b_ref, o_ref):
    a = jnp.concatenate([agg_ref[0], agg_ref[1]], axis=1)
    deg = deg_ref[0] + deg_ref[1]
    deg = jnp.maximum(deg[:, 0:1], 1.0)
    h = jnp.maximum(
        jnp.dot(a / deg, w_ref[...], preferred_element_type=jnp.float32)
        + b_ref[...], 0.0)
    o_ref[0, :, :] = h[:, :DH]
    o_ref[1, :, :] = h[:, DH:]


def _tc2_body(agg_ref, deg_ref, w2, b2, wa1, ba1, wa2, ba2,
              wv1, bv1, wv2, bv2, q_ref, acc_ref):
    i = pl.program_id(0)
    a = jnp.concatenate([agg_ref[0], agg_ref[1]], axis=1)
    deg = deg_ref[0] + deg_ref[1]
    deg = jnp.maximum(deg[:, 0:1], 1.0)
    h = jnp.maximum(
        jnp.dot(a / deg, w2[...], preferred_element_type=jnp.float32)
        + b2[...], 0.0)
    part = jnp.sum(h, axis=0, keepdims=True)

    @pl.when(i == 0)
    def _():
        acc_ref[...] = part

    @pl.when(i > 0)
    def _():
        acc_ref[...] = acc_ref[...] + part

    @pl.when(i == pl.num_programs(0) - 1)
    def _():
        ge = acc_ref[...] * (1.0 / N_NODES)
        adv = jnp.maximum(
            jnp.dot(ge, wa1[...], preferred_element_type=jnp.float32)
            + ba1[...], 0.0)
        aq = (jnp.dot(adv, wa2[...], preferred_element_type=jnp.float32)
              + ba2[...])
        val = jnp.maximum(
            jnp.dot(ge, wv1[...], preferred_element_type=jnp.float32)
            + bv1[...], 0.0)
        v = (jnp.dot(val, wv2[...], preferred_element_type=jnp.float32)
             + bv2[...])
        q_ref[...] = v + aq - jnp.mean(aq)


_tc_layer1 = pl.pallas_call(
    _tc1_body,
    grid=(GRID,),
    in_specs=[
        pl.BlockSpec((2, BLK, DH), lambda i: (0, i, 0)),
        pl.BlockSpec((2, BLK, 16), lambda i: (0, i, 0)),
        pl.BlockSpec((D, D), lambda i: (0, 0)),
        pl.BlockSpec((1, D), lambda i: (0, 0)),
    ],
    out_specs=pl.BlockSpec((2, BLK, DH), lambda i: (0, i, 0)),
    out_shape=jax.ShapeDtypeStruct((2, N_NODES, DH), jnp.float32),
)

_tc_head = pl.pallas_call(
    _tc2_body,
    grid=(GRID,),
    in_specs=[
        pl.BlockSpec((2, BLK, DH), lambda i: (0, i, 0)),
        pl.BlockSpec((2, BLK, 16), lambda i: (0, i, 0)),
        pl.BlockSpec((D, D), lambda i: (0, 0)),
        pl.BlockSpec((1, D), lambda i: (0, 0)),
        pl.BlockSpec((D, D_STREAM), lambda i: (0, 0)),
        pl.BlockSpec((1, D_STREAM), lambda i: (0, 0)),
        pl.BlockSpec((D_STREAM, N_ACTIONS), lambda i: (0, 0)),
        pl.BlockSpec((1, N_ACTIONS), lambda i: (0, 0)),
        pl.BlockSpec((D, D_STREAM), lambda i: (0, 0)),
        pl.BlockSpec((1, D_STREAM), lambda i: (0, 0)),
        pl.BlockSpec((D_STREAM, 1), lambda i: (0, 0)),
        pl.BlockSpec((1, 1), lambda i: (0, 0)),
    ],
    out_specs=pl.BlockSpec((1, N_ACTIONS), lambda i: (0, 0)),
    out_shape=jax.ShapeDtypeStruct((1, N_ACTIONS), jnp.float32),
    scratch_shapes=[pltpu.VMEM((1, D), jnp.float32)],
)


def kernel(x, edge_index, W1, b1, W2, b2, Wa1, ba1, Wa2, ba2,
           Wv1, bv1, Wv2, bv2):
    src = edge_index[0].astype(jnp.int32)
    dst = edge_index[1].astype(jnp.int32)
    pad = E_PAD - N_EDGES
    srcq = jnp.concatenate([src, jnp.zeros((pad,), jnp.int32)])
    dstq = jnp.concatenate([dst, jnp.full((pad,), DUMMY, jnp.int32)])
    srcq = srcq.reshape(NROWS_TOT, CHUNK)
    dstq = dstq.reshape(NROWS_TOT, CHUNK)
    srcq2 = jnp.stack([srcq, srcq + N_NODES])
    z64 = jnp.zeros((AGG_ROWS, DH), jnp.float32)
    z16 = jnp.zeros((AGG_ROWS, 16), jnp.float32)
    ones16 = jnp.ones((CHUNK, 16), jnp.float32)
    x_cat = jnp.concatenate([x[:, :DH], x[:, DH:]], axis=0)

    agg1, degm = _sc_layer1(x_cat, srcq2, dstq, z64, z16, ones16)
    h1 = _tc_layer1(agg1, degm, W1, b1.reshape(1, D))
    h1_cat = h1.reshape(2 * N_NODES, DH)
    (agg2,) = _sc_layer2(h1_cat, srcq2, dstq, z64)
    q = _tc_head(agg2, degm, W2, b2.reshape(1, D),
                 Wa1, ba1.reshape(1, D_STREAM), Wa2, ba2.reshape(1, N_ACTIONS),
                 Wv1, bv1.reshape(1, D_STREAM), Wv2, bv2.reshape(1, 1))
    return q


# speedup vs baseline: 5.3502x; 1.0007x over previous
import jax
import jax.numpy as jnp
from jax import lax
from jax.experimental import pallas as pl
from jax.experimental.pallas import tpu as pltpu
from jax.experimental.pallas import tpu_sc as plsc

N_NODES = 10000
N_EDGES = 320000
D = 128
DH = D // 2
D_STREAM = 256
N_ACTIONS = 64

NC, NS = 2, 16
CHUNK = 128
RPW = 160
NROWS_TOT = RPW * NS
E_PAD = NROWS_TOT * CHUNK
DUMMY = N_NODES
AGG_ROWS = 10112
ZROWS = AGG_ROWS // NS
DEG_RPW = RPW // NC
DEGW = 8

_MESH = plsc.VectorSubcoreMesh(core_axis_name="c", subcore_axis_name="s")


NBUF = 5


def _sc1_body(x_hbm, srcq, dstq, z64, z16, ones_hbm, agg_out, deg_out,
              src_v, dst_v, r0, r1, r2, r3, r4, ones_v,
              agg_sh, deg_sh, g0, g1, g2, g3, g4):
    rows = (r0, r1, r2, r3, r4)
    sems = (g0, g1, g2, g3, g4)
    c = lax.axis_index("c")
    s = lax.axis_index("s")
    pltpu.sync_copy(z64.at[pl.ds(s * ZROWS, ZROWS)],
                    agg_sh.at[pl.ds(s * ZROWS, ZROWS)])
    pltpu.sync_copy(z16.at[pl.ds(s * ZROWS, ZROWS)],
                    deg_sh.at[pl.ds(s * ZROWS, ZROWS)])
    pltpu.sync_copy(ones_hbm, ones_v)
    pltpu.sync_copy(srcq.at[c, pl.ds(s * RPW, RPW)], src_v)
    pltpu.sync_copy(dstq.at[pl.ds(s * RPW, RPW)], dst_v)
    plsc.subcore_barrier()

    deg_lo = c * DEG_RPW
    for b in range(NBUF):
        pltpu.async_copy(x_hbm.at[src_v.at[b]], rows[b], sems[b])

    def group(g, carry):
        for b in range(NBUF):
            j = g * NBUF + b
            pltpu.make_async_copy(x_hbm.at[src_v.at[j]],
                                  rows[b], sems[b]).wait()
            pltpu.sync_copy(rows[b], agg_sh.at[dst_v.at[j]], add=True)

            @pl.when((j >= deg_lo) & (j < deg_lo + DEG_RPW))
            def _():
                pltpu.sync_copy(ones_v, deg_sh.at[dst_v.at[j]], add=True)

            @pl.when(j + NBUF < RPW)
            def _():
                pltpu.async_copy(x_hbm.at[src_v.at[j + NBUF]],
                                 rows[b], sems[b])

        return carry

    lax.fori_loop(0, RPW // NBUF, group, 0)
    plsc.subcore_barrier()
    pltpu.sync_copy(agg_sh.at[pl.ds(s * ZROWS, ZROWS)],
                    agg_out.at[c, pl.ds(s * ZROWS, ZROWS)])
    pltpu.sync_copy(deg_sh.at[pl.ds(s * ZROWS, ZROWS)],
                    deg_out.at[c, pl.ds(s * ZROWS, ZROWS)])


def _sc2_body(h_hbm, srcq, dstq, z64, agg_out,
              src_v, dst_v, r0, r1, r2, r3, r4, agg_sh,
              g0, g1, g2, g3, g4):
    rows = (r0, r1, r2, r3, r4)
    sems = (g0, g1, g2, g3, g4)
    c = lax.axis_index("c")
    s = lax.axis_index("s")
    pltpu.sync_copy(z64.at[pl.ds(s * ZROWS, ZROWS)],
                    agg_sh.at[pl.ds(s * ZROWS, ZROWS)])
    pltpu.sync_copy(srcq.at[c, pl.ds(s * RPW, RPW)], src_v)
    pltpu.sync_copy(dstq.at[pl.ds(s * RPW, RPW)], dst_v)
    plsc.subcore_barrier()

    for b in range(NBUF):
        pltpu.async_copy(h_hbm.at[src_v.at[b]], rows[b], sems[b])

    def group(g, carry):
        for b in range(NBUF):
            j = g * NBUF + b
            pltpu.make_async_copy(h_hbm.at[src_v.at[j]],
                                  rows[b], sems[b]).wait()
            pltpu.sync_copy(rows[b], agg_sh.at[dst_v.at[j]], add=True)

            @pl.when(j + NBUF < RPW)
            def _():
                pltpu.async_copy(h_hbm.at[src_v.at[j + NBUF]],
                                 rows[b], sems[b])

        return carry

    lax.fori_loop(0, RPW // NBUF, group, 0)
    plsc.subcore_barrier()
    pltpu.sync_copy(agg_sh.at[pl.ds(s * ZROWS, ZROWS)],
                    agg_out.at[c, pl.ds(s * ZROWS, ZROWS)])


_sc_layer1 = pl.kernel(
    _sc1_body,
    out_type=[
        jax.ShapeDtypeStruct((NC, AGG_ROWS, DH), jnp.float32),
        jax.ShapeDtypeStruct((NC, AGG_ROWS, DEGW), jnp.float32),
    ],
    mesh=_MESH,
    compiler_params=pltpu.CompilerParams(use_tc_tiling_on_sc=False),
    scratch_types=[
        pltpu.VMEM((RPW, CHUNK), jnp.int32),
        pltpu.VMEM((RPW, CHUNK), jnp.int32),
    ] + [pltpu.VMEM((CHUNK, DH), jnp.float32)] * NBUF + [
        pltpu.VMEM((CHUNK, DEGW), jnp.float32),
        pltpu.VMEM_SHARED((AGG_ROWS, DH), jnp.float32),
        pltpu.VMEM_SHARED((AGG_ROWS, DEGW), jnp.float32),
    ] + [pltpu.SemaphoreType.DMA] * NBUF,
)

_sc_layer2 = pl.kernel(
    _sc2_body,
    out_type=[jax.ShapeDtypeStruct((NC, AGG_ROWS, DH), jnp.float32)],
    mesh=_MESH,
    compiler_params=pltpu.CompilerParams(use_tc_tiling_on_sc=False),
    scratch_types=[
        pltpu.VMEM((RPW, CHUNK), jnp.int32),
        pltpu.VMEM((RPW, CHUNK), jnp.int32),
    ] + [pltpu.VMEM((CHUNK, DH), jnp.float32)] * NBUF + [
        pltpu.VMEM_SHARED((AGG_ROWS, DH), jnp.float32),
    ] + [pltpu.SemaphoreType.DMA] * NBUF,
)

BLK = 2000
GRID = N_NODES // BLK


def _tc1_body(agg_ref, deg_ref, w_ref, b_ref, o_ref):
    a = jnp.concatenate([agg_ref[0], agg_ref[1]], axis=1)
    deg = deg_ref[0] + deg_ref[1]
    deg = jnp.maximum(deg[:, 0:1], 1.0)
    h = jnp.maximum(
        jnp.dot(a / deg, w_ref[...], preferred_element_type=jnp.float32)
        + b_ref[...], 0.0)
    o_ref[0, :, :] = h[:, :DH]
    o_ref[1, :, :] = h[:, DH:]


def _tc2_body(agg_ref, deg_ref, w2, b2, wa1, ba1, wa2, ba2,
              wv1, bv1, wv2, bv2, q_ref, acc_ref):
    i = pl.program_id(0)
    a = jnp.concatenate([agg_ref[0], agg_ref[1]], axis=1)
    deg = deg_ref[0] + deg_ref[1]
    deg = jnp.maximum(deg[:, 0:1], 1.0)
    h = jnp.maximum(
        jnp.dot(a / deg, w2[...], preferred_element_type=jnp.float32)
        + b2[...], 0.0)
    part = jnp.sum(h, axis=0, keepdims=True)

    @pl.when(i == 0)
    def _():
        acc_ref[...] = part

    @pl.when(i > 0)
    def _():
        acc_ref[...] = acc_ref[...] + part

    @pl.when(i == pl.num_programs(0) - 1)
    def _():
        ge = acc_ref[...] * (1.0 / N_NODES)
        adv = jnp.maximum(
            jnp.dot(ge, wa1[...], preferred_element_type=jnp.float32)
            + ba1[...], 0.0)
        aq = (jnp.dot(adv, wa2[...], preferred_element_type=jnp.float32)
              + ba2[...])
        val = jnp.maximum(
            jnp.dot(ge, wv1[...], preferred_element_type=jnp.float32)
            + bv1[...], 0.0)
        v = (jnp.dot(val, wv2[...], preferred_element_type=jnp.float32)
             + bv2[...])
        q_ref[...] = v + aq - jnp.mean(aq)


_tc_layer1 = pl.pallas_call(
    _tc1_body,
    grid=(GRID,),
    in_specs=[
        pl.BlockSpec((2, BLK, DH), lambda i: (0, i, 0)),
        pl.BlockSpec((2, BLK, DEGW), lambda i: (0, i, 0)),
        pl.BlockSpec((D, D), lambda i: (0, 0)),
        pl.BlockSpec((1, D), lambda i: (0, 0)),
    ],
    out_specs=pl.BlockSpec((2, BLK, DH), lambda i: (0, i, 0)),
    out_shape=jax.ShapeDtypeStruct((2, N_NODES, DH), jnp.float32),
)

_tc_head = pl.pallas_call(
    _tc2_body,
    grid=(GRID,),
    in_specs=[
        pl.BlockSpec((2, BLK, DH), lambda i: (0, i, 0)),
        pl.BlockSpec((2, BLK, DEGW), lambda i: (0, i, 0)),
        pl.BlockSpec((D, D), lambda i: (0, 0)),
        pl.BlockSpec((1, D), lambda i: (0, 0)),
        pl.BlockSpec((D, D_STREAM), lambda i: (0, 0)),
        pl.BlockSpec((1, D_STREAM), lambda i: (0, 0)),
        pl.BlockSpec((D_STREAM, N_ACTIONS), lambda i: (0, 0)),
        pl.BlockSpec((1, N_ACTIONS), lambda i: (0, 0)),
        pl.BlockSpec((D, D_STREAM), lambda i: (0, 0)),
        pl.BlockSpec((1, D_STREAM), lambda i: (0, 0)),
        pl.BlockSpec((D_STREAM, 1), lambda i: (0, 0)),
        pl.BlockSpec((1, 1), lambda i: (0, 0)),
    ],
    out_specs=pl.BlockSpec((1, N_ACTIONS), lambda i: (0, 0)),
    out_shape=jax.ShapeDtypeStruct((1, N_ACTIONS), jnp.float32),
    scratch_shapes=[pltpu.VMEM((1, D), jnp.float32)],
)


def kernel(x, edge_index, W1, b1, W2, b2, Wa1, ba1, Wa2, ba2,
           Wv1, bv1, Wv2, bv2):
    src = edge_index[0].astype(jnp.int32)
    dst = edge_index[1].astype(jnp.int32)
    pad = E_PAD - N_EDGES
    srcq = jnp.concatenate([src, jnp.zeros((pad,), jnp.int32)])
    dstq = jnp.concatenate([dst, jnp.full((pad,), DUMMY, jnp.int32)])
    srcq = srcq.reshape(NROWS_TOT, CHUNK)
    dstq = dstq.reshape(NROWS_TOT, CHUNK)
    srcq2 = jnp.stack([srcq, srcq + N_NODES])
    z64 = jnp.zeros((AGG_ROWS, DH), jnp.float32)
    z16 = jnp.zeros((AGG_ROWS, DEGW), jnp.float32)
    ones16 = jnp.ones((CHUNK, DEGW), jnp.float32)
    x_cat = jnp.concatenate([x[:, :DH], x[:, DH:]], axis=0)

    agg1, degm = _sc_layer1(x_cat, srcq2, dstq, z64, z16, ones16)
    h1 = _tc_layer1(agg1, degm, W1, b1.reshape(1, D))
    h1_cat = h1.reshape(2 * N_NODES, DH)
    (agg2,) = _sc_layer2(h1_cat, srcq2, dstq, z64)
    q = _tc_head(agg2, degm, W2, b2.reshape(1, D),
                 Wa1, ba1.reshape(1, D_STREAM), Wa2, ba2.reshape(1, N_ACTIONS),
                 Wv1, bv1.reshape(1, D_STREAM), Wv2, bv2.reshape(1, 1))
    return q
